# baseline (speedup 1.0000x reference)
import functools
import math

import jax
import jax.numpy as jnp
from jax import lax
from jax.experimental import pallas as pl
from jax.experimental.pallas import tpu as pltpu
from jax.experimental.pallas import tpu_sc as plsc

D_MODEL = 64
SCALE = 4.0
CHUNK = 128
NBUF = 4
ROW_UNROLL = 4


def _embed_lookup(ids1d, table):
    n_rows = ids1d.shape[0]
    info = plsc.get_sparse_core_info()
    nw = info.num_cores * info.num_subcores
    ipw = n_rows // nw
    n_chunks = ipw // CHUNK
    n_outer = n_chunks // NBUF
    assert n_chunks % NBUF == 0 and n_outer >= 2

    mesh = plsc.VectorSubcoreMesh(core_axis_name="c", subcore_axis_name="s")

    @functools.partial(
        pl.kernel,
        mesh=mesh,
        out_type=jax.ShapeDtypeStruct((n_rows, D_MODEL), jnp.float32),
        scratch_types=[
            pltpu.VMEM((ipw,), jnp.int32),
            pltpu.VMEM((NBUF, CHUNK, D_MODEL), jnp.float32),
            pltpu.VMEM((2, CHUNK, D_MODEL), jnp.float32),
            [pltpu.SemaphoreType.DMA] * NBUF,
            [pltpu.SemaphoreType.DMA] * 2,
        ],
        compiler_params=pltpu.CompilerParams(use_tc_tiling_on_sc=False),
    )
    def body(table_hbm, ids_hbm, out_hbm, idx_v, g_bufs, o_bufs, gsems, osems):
        wid = lax.axis_index("s") * info.num_cores + lax.axis_index("c")
        idx0 = wid * ipw
        pltpu.sync_copy(ids_hbm.at[pl.ds(idx0, ipw)], idx_v)

        def gather_args(j, b):
            idx = idx_v.at[pl.ds(j * CHUNK, CHUNK)]
            return table_hbm.at[idx], g_bufs.at[b]

        def issue_gather(j, b):
            src, dst = gather_args(j, b)
            pltpu.async_copy(src, dst, gsems[b])

        def wait_gather(j, b):
            src, dst = gather_args(j, b)
            pltpu.make_async_copy(src, dst, gsems[b]).wait()

        def out_args(j, b):
            src = o_bufs.at[b % 2]
            return src, out_hbm.at[pl.ds((idx0 + j * CHUNK), CHUNK)]

        def issue_out(j, b):
            src, dst = out_args(j, b)
            pltpu.async_copy(src, dst, osems[b % 2])

        def wait_out(j, b):
            src, dst = out_args(j, b)
            pltpu.make_async_copy(src, dst, osems[b % 2]).wait()

        def scale(b):
            src = g_bufs.at[b]
            dst = o_bufs.at[b % 2]

            def rows(i, r0):
                for ru in range(ROW_UNROLL):
                    for c in range(D_MODEL // 16):
                        sl = pl.ds(c * 16, 16)
                        dst[r0 + ru, sl] = src[r0 + ru, sl] * SCALE
                return r0 + ROW_UNROLL

            lax.fori_loop(0, CHUNK // ROW_UNROLL, rows, 0)

        for b in range(NBUF):
            issue_gather(b, b)

        for b in range(NBUF):
            wait_gather(b, b)
            if b >= 2:
                wait_out(b - 2, b - 2)
            scale(b)
            issue_gather(NBUF + b, b)
            issue_out(b, b)

        def group(g, _):
            for b in range(NBUF):
                j = g * NBUF + b
                wait_gather(j, b)
                wait_out(j - 2, b - 2 if b >= 2 else b + 2)
                scale(b)
                issue_gather(j + NBUF, b)
                issue_out(j, b)
            return 0

        lax.fori_loop(1, n_outer - 1, group, 0)

        for b in range(NBUF):
            j = (n_outer - 1) * NBUF + b
            wait_gather(j, b)
            wait_out(j - 2, b - 2 if b >= 2 else b + 2)
            scale(b)
            issue_out(j, b)

        wait_out(n_chunks - 2, 2)
        wait_out(n_chunks - 1, 3)

    return body(table, ids1d)


def kernel(input, table):
    b, s = input.shape
    out = _embed_lookup(input.reshape(-1), table * 2.0)
    return out.reshape(b, s, D_MODEL)


# speedup vs baseline: 1.0558x; 1.0558x over previous
import functools
import math

import jax
import jax.numpy as jnp
from jax import lax
from jax.experimental import pallas as pl
from jax.experimental.pallas import tpu as pltpu
from jax.experimental.pallas import tpu_sc as plsc

D_MODEL = 64
SCALE = math.sqrt(D_MODEL)
RAW = 128
CHUNK = 128
NBUF = 4
ROW_UNROLL = 4


def _embed_lookup(idshalf, paroff, tpairs):
    n_rows = idshalf.shape[0]
    info = plsc.get_sparse_core_info()
    nw = info.num_cores * info.num_subcores
    ipw = n_rows // nw
    n_chunks = ipw // CHUNK
    n_outer = n_chunks // NBUF
    assert n_chunks % NBUF == 0 and n_outer >= 2

    mesh = plsc.VectorSubcoreMesh(core_axis_name="c", subcore_axis_name="s")

    @functools.partial(
        pl.kernel,
        mesh=mesh,
        out_type=jax.ShapeDtypeStruct((n_rows, D_MODEL), jnp.float32),
        scratch_types=[
            pltpu.VMEM((ipw,), jnp.int32),
            pltpu.VMEM((NBUF, CHUNK, RAW), jnp.float32),
            pltpu.VMEM((2, CHUNK, D_MODEL), jnp.float32),
            pltpu.VMEM((NBUF, CHUNK), jnp.int32),
            [pltpu.SemaphoreType.DMA] * NBUF,
            [pltpu.SemaphoreType.DMA] * 2,
            [pltpu.SemaphoreType.DMA] * NBUF,
        ],
        compiler_params=pltpu.CompilerParams(needs_layout_passes=False),
    )
    def body(tp_hbm, ids_hbm, par_hbm, out_hbm, idx_v, g_bufs, o_bufs,
             par_v, gsems, osems, psems):
        wid = lax.axis_index("s") * info.num_cores + lax.axis_index("c")
        idx0 = wid * ipw
        pltpu.sync_copy(ids_hbm.at[pl.ds(idx0, ipw)], idx_v)

        def gather_args(j, b):
            idx = idx_v.at[pl.ds(j * CHUNK, CHUNK)]
            return tp_hbm.at[idx], g_bufs.at[b]

        def issue_gather(j, b):
            src, dst = gather_args(j, b)
            pltpu.async_copy(src, dst, gsems[b])
            pltpu.async_copy(
                par_hbm.at[pl.ds(idx0 + j * CHUNK, CHUNK)], par_v.at[b],
                psems[b],
            )

        def wait_gather(j, b):
            src, dst = gather_args(j, b)
            pltpu.make_async_copy(src, dst, gsems[b]).wait()
            pltpu.make_async_copy(
                par_hbm.at[pl.ds(idx0 + j * CHUNK, CHUNK)], par_v.at[b],
                psems[b],
            ).wait()

        def out_args(j, b):
            src = o_bufs.at[b % 2]
            return src, out_hbm.at[pl.ds((idx0 + j * CHUNK), CHUNK)]

        def issue_out(j, b):
            src, dst = out_args(j, b)
            pltpu.async_copy(src, dst, osems[b % 2])

        def wait_out(j, b):
            src, dst = out_args(j, b)
            pltpu.make_async_copy(src, dst, osems[b % 2]).wait()

        lanes = lax.iota(jnp.int32, 16)

        def scale(b):
            src = g_bufs.at[b]
            dst = o_bufs.at[b % 2]
            par_c = par_v.at[b]

            def rows(i, r0):
                for ru in range(ROW_UNROLL):
                    r = r0 + ru
                    rsplat = jnp.full((16,), 0, jnp.int32) + r
                    offv = plsc.load_gather(par_c, [rsplat]) + lanes
                    for c in range(D_MODEL // 16):
                        v = plsc.load_gather(src, [rsplat, offv + (c * 16)])
                        dst[r, pl.ds(c * 16, 16)] = v * SCALE
                return r0 + ROW_UNROLL

            lax.fori_loop(0, CHUNK // ROW_UNROLL, rows, 0)

        for b in range(NBUF):
            issue_gather(b, b)

        for b in range(NBUF):
            wait_gather(b, b)
            if b >= 2:
                wait_out(b - 2, b - 2)
            scale(b)
            issue_gather(NBUF + b, b)
            issue_out(b, b)

        def group(g, _):
            for b in range(NBUF):
                j = g * NBUF + b
                wait_gather(j, b)
                wait_out(j - 2, b - 2 if b >= 2 else b + 2)
                scale(b)
                issue_gather(j + NBUF, b)
                issue_out(j, b)
            return 0

        lax.fori_loop(1, n_outer - 1, group, 0)

        for b in range(NBUF):
            j = (n_outer - 1) * NBUF + b
            wait_gather(j, b)
            wait_out(j - 2, b - 2 if b >= 2 else b + 2)
            scale(b)
            issue_out(j, b)

        wait_out(n_chunks - 2, 2)
        wait_out(n_chunks - 1, 3)

    return body(tpairs, idshalf, paroff)


def kernel(input, table):
    b, s = input.shape
    v = table.shape[0]
    ids1d = input.reshape(-1)
    tpairs = table.reshape(v // 2, RAW)
    out = _embed_lookup(ids1d >> 1, (ids1d & 1) * D_MODEL, tpairs)
    return out.reshape(b, s, D_MODEL)


# speedup vs baseline: 1.5326x; 1.4517x over previous
import functools
import math

import jax
import jax.numpy as jnp
from jax import lax
from jax.experimental import pallas as pl
from jax.experimental.pallas import tpu as pltpu
from jax.experimental.pallas import tpu_sc as plsc

D_MODEL = 64
SCALE = math.sqrt(D_MODEL)
RAW = 128
CHUNK = 128
NBUF = 4
ROW_UNROLL = 4


def _embed_lookup(ids1d, tpad):
    n_rows = ids1d.shape[0]
    info = plsc.get_sparse_core_info()
    nw = info.num_cores * info.num_subcores
    ipw = n_rows // nw
    n_chunks = ipw // CHUNK
    n_outer = n_chunks // NBUF
    assert n_chunks % NBUF == 0 and n_outer >= 2

    mesh = plsc.VectorSubcoreMesh(core_axis_name="c", subcore_axis_name="s")

    @functools.partial(
        pl.kernel,
        mesh=mesh,
        out_type=jax.ShapeDtypeStruct((n_rows, D_MODEL), jnp.float32),
        scratch_types=[
            pltpu.VMEM((ipw,), jnp.int32),
            pltpu.VMEM((NBUF, CHUNK, RAW), jnp.float32),
            pltpu.VMEM((2, CHUNK, D_MODEL), jnp.float32),
            [pltpu.SemaphoreType.DMA] * NBUF,
            [pltpu.SemaphoreType.DMA] * 2,
        ],
    )
    def body(tpad_hbm, ids_hbm, out_hbm, idx_v, g_bufs, o_bufs, gsems, osems):
        wid = lax.axis_index("s") * info.num_cores + lax.axis_index("c")
        idx0 = wid * ipw
        pltpu.sync_copy(ids_hbm.at[pl.ds(idx0, ipw)], idx_v)

        def gather_args(j, b):
            idx = idx_v.at[pl.ds(j * CHUNK, CHUNK)]
            return tpad_hbm.at[idx], g_bufs.at[b]

        def issue_gather(j, b):
            src, dst = gather_args(j, b)
            pltpu.async_copy(src, dst, gsems[b])

        def wait_gather(j, b):
            src, dst = gather_args(j, b)
            pltpu.make_async_copy(src, dst, gsems[b]).wait()

        def out_args(j, b):
            src = o_bufs.at[b % 2]
            return src, out_hbm.at[pl.ds((idx0 + j * CHUNK), CHUNK)]

        def issue_out(j, b):
            src, dst = out_args(j, b)
            pltpu.async_copy(src, dst, osems[b % 2])

        def wait_out(j, b):
            src, dst = out_args(j, b)
            pltpu.make_async_copy(src, dst, osems[b % 2]).wait()

        def scale(b):
            src = g_bufs.at[b]
            dst = o_bufs.at[b % 2]

            def rows(i, r0):
                for ru in range(ROW_UNROLL):
                    for c in range(D_MODEL // 16):
                        sl = pl.ds(c * 16, 16)
                        dst[r0 + ru, sl] = src[r0 + ru, sl] * SCALE
                return r0 + ROW_UNROLL

            lax.fori_loop(0, CHUNK // ROW_UNROLL, rows, 0)

        for b in range(NBUF):
            issue_gather(b, b)

        for b in range(NBUF):
            wait_gather(b, b)
            if b >= 2:
                wait_out(b - 2, b - 2)
            scale(b)
            issue_gather(NBUF + b, b)
            issue_out(b, b)

        def group(g, _):
            for b in range(NBUF):
                j = g * NBUF + b
                wait_gather(j, b)
                wait_out(j - 2, b - 2 if b >= 2 else b + 2)
                scale(b)
                issue_gather(j + NBUF, b)
                issue_out(j, b)
            return 0

        lax.fori_loop(1, n_outer - 1, group, 0)

        for b in range(NBUF):
            j = (n_outer - 1) * NBUF + b
            wait_gather(j, b)
            wait_out(j - 2, b - 2 if b >= 2 else b + 2)
            scale(b)
            issue_out(j, b)

        wait_out(n_chunks - 2, 2)
        wait_out(n_chunks - 1, 3)

    return body(tpad, ids1d)


def kernel(input, table):
    b, s = input.shape
    tpad = jnp.pad(table, ((0, 0), (0, RAW - D_MODEL)))
    out = _embed_lookup(input.reshape(-1), tpad)
    return out.reshape(b, s, D_MODEL)
